# baseline (speedup 1.0000x reference)
import jax
import jax.numpy as jnp
from jax.experimental import pallas as pl


def _gat_kernel(x_ref, w_ref, asrc_ref, adst_ref, bias_ref, o_ref):
    h = jax.lax.dot_general(
        x_ref[:], w_ref[:], (((1,), (1,)), ((), ())),
        preferred_element_type=jnp.float32)
    s = jax.lax.dot_general(
        asrc_ref[:], h, (((1,), (1,)), ((), ())),
        preferred_element_type=jnp.float32)
    d = jnp.dot(h, adst_ref[:], preferred_element_type=jnp.float32)
    logits = d + s
    logits = jnp.maximum(logits, 0.2 * logits)
    m = jnp.max(logits, axis=1, keepdims=True)
    e = jnp.exp(logits - m)
    ones = jnp.ones((e.shape[1], 1), jnp.float32)
    den = jnp.dot(e, ones, preferred_element_type=jnp.float32)
    num = jnp.dot(e, h, preferred_element_type=jnp.float32)
    o_ref[:] = num / (den + 1e-16) + bias_ref[:]


def kernel(x, W, att_src, att_dst, bias, edge_index):
    b, _, nc, nf = x.shape
    n = b * nc
    xf = x.reshape(n, nf)
    out = pl.pallas_call(
        _gat_kernel,
        out_shape=jax.ShapeDtypeStruct((n, nf), jnp.float32),
    )(xf, W, att_src.reshape(1, nf), att_dst.reshape(nf, 1),
      bias.reshape(1, nf))
    return out.reshape(b, nc, nf)[:, None, :, :]


# speedup vs baseline: 1.0641x; 1.0641x over previous
import jax
import jax.numpy as jnp
from jax.experimental import pallas as pl


def _gat_kernel(x_ref, w_ref, asrc_ref, adst_ref, bias_ref, o_ref):
    h = jax.lax.dot_general(
        x_ref[:], w_ref[:], (((1,), (1,)), ((), ())),
        preferred_element_type=jnp.float32)
    s = jax.lax.dot_general(
        asrc_ref[:], h, (((1,), (1,)), ((), ())),
        preferred_element_type=jnp.float32)
    d = jnp.dot(h, adst_ref[:], preferred_element_type=jnp.float32)
    logits = d + s
    logits = jnp.maximum(logits, 0.2 * logits)
    m = jnp.max(logits, axis=1, keepdims=True)
    e = jnp.exp(logits - m)
    den = jnp.sum(e, axis=1, keepdims=True)
    num = jnp.dot(e, h, preferred_element_type=jnp.float32)
    o_ref[:] = num / (den + 1e-16) + bias_ref[:]


def kernel(x, W, att_src, att_dst, bias, edge_index):
    b, _, nc, nf = x.shape
    n = b * nc
    xf = x.reshape(n, nf)
    out = pl.pallas_call(
        _gat_kernel,
        out_shape=jax.ShapeDtypeStruct((n, nf), jnp.float32),
    )(xf, W, att_src.reshape(1, nf), att_dst.reshape(nf, 1),
      bias.reshape(1, nf))
    return out.reshape(b, nc, nf)[:, None, :, :]


# speedup vs baseline: 1.2697x; 1.1932x over previous
import jax
import jax.numpy as jnp
from jax.experimental import pallas as pl


def _gat_kernel(x_ref, w_ref, asrc_ref, adst_ref, bias_ref, o_ref):
    xm = x_ref[0, 0]
    h = jax.lax.dot_general(
        xm, w_ref[:], (((1,), (1,)), ((), ())),
        preferred_element_type=jnp.float32)
    s = jax.lax.dot_general(
        asrc_ref[:], h, (((1,), (1,)), ((), ())),
        preferred_element_type=jnp.float32)
    d = jax.lax.dot_general(
        h, adst_ref[:], (((1,), (1,)), ((), ())),
        preferred_element_type=jnp.float32)
    logits = d + s
    logits = jnp.maximum(logits, 0.2 * logits)
    m = jnp.max(logits, axis=1, keepdims=True)
    e = jnp.exp(logits - m)
    den = jnp.sum(e, axis=1, keepdims=True)
    num = jnp.dot(e, h, preferred_element_type=jnp.float32)
    o_ref[0, 0] = num / (den + 1e-16) + bias_ref[:]


def kernel(x, W, att_src, att_dst, bias, edge_index):
    b, _, nc, nf = x.shape
    return pl.pallas_call(
        _gat_kernel,
        out_shape=jax.ShapeDtypeStruct((b, 1, nc, nf), jnp.float32),
    )(x, W, att_src.reshape(1, nf), att_dst.reshape(1, nf),
      bias.reshape(1, nf))


# speedup vs baseline: 1.2746x; 1.0039x over previous
import jax
import jax.numpy as jnp
from jax.experimental import pallas as pl


def _gat_kernel(x_ref, w_ref, asrc_ref, adst_ref, bias_ref, o_ref):
    xm = x_ref[0, 0]
    asrc = asrc_ref[:].reshape(1, -1)
    adst = adst_ref[:].reshape(1, -1)
    bias = bias_ref[:].reshape(1, -1)
    h = jax.lax.dot_general(
        xm, w_ref[:], (((1,), (1,)), ((), ())),
        preferred_element_type=jnp.float32)
    s = jax.lax.dot_general(
        asrc, h, (((1,), (1,)), ((), ())),
        preferred_element_type=jnp.float32)
    d = jax.lax.dot_general(
        h, adst, (((1,), (1,)), ((), ())),
        preferred_element_type=jnp.float32)
    logits = d + s
    logits = jnp.maximum(logits, 0.2 * logits)
    m = jnp.max(logits, axis=1, keepdims=True)
    e = jnp.exp(logits - m)
    den = jnp.sum(e, axis=1, keepdims=True)
    num = jnp.dot(e, h, preferred_element_type=jnp.float32)
    o_ref[0, 0] = num / (den + 1e-16) + bias


def kernel(x, W, att_src, att_dst, bias, edge_index):
    b, _, nc, nf = x.shape
    return pl.pallas_call(
        _gat_kernel,
        out_shape=jax.ShapeDtypeStruct((b, 1, nc, nf), jnp.float32),
    )(x, W, att_src, att_dst, bias)


# speedup vs baseline: 2.7386x; 2.1486x over previous
import jax
import jax.numpy as jnp
from jax.experimental import pallas as pl


def _gat_kernel(x_ref, w_ref, asrc_ref, adst_ref, bias_ref, o_ref):
    xt = x_ref[0, 0]
    asrc = asrc_ref[:].reshape(1, -1)
    adst = adst_ref[:].reshape(1, -1)
    ht = jax.lax.dot_general(
        w_ref[:], xt, (((1,), (0,)), ((), ())),
        preferred_element_type=jnp.float32)
    s = jax.lax.dot_general(
        asrc, ht, (((1,), (0,)), ((), ())),
        preferred_element_type=jnp.float32)
    d = jax.lax.dot_general(
        adst, ht, (((1,), (0,)), ((), ())),
        preferred_element_type=jnp.float32)
    s_col = s.reshape(-1, 1)
    logits = s_col + d
    logits = jnp.maximum(logits, 0.2 * logits)
    m = jnp.max(logits, axis=0, keepdims=True)
    e = jnp.exp(logits - m)
    den = jnp.sum(e, axis=0, keepdims=True)
    num = jax.lax.dot_general(
        ht, e, (((1,), (0,)), ((), ())),
        preferred_element_type=jnp.float32)
    bias_col = bias_ref[:].reshape(-1, 1)
    o_ref[0, 0] = num / (den + 1e-16) + bias_col


def kernel(x, W, att_src, att_dst, bias, edge_index):
    b, _, nc, nf = x.shape
    out_t = pl.pallas_call(
        _gat_kernel,
        out_shape=jax.ShapeDtypeStruct((b, 1, nf, nc), jnp.float32),
    )(jnp.swapaxes(x, 2, 3), W, att_src, att_dst, bias)
    return jnp.swapaxes(out_t, 2, 3)
